# baseline (speedup 1.0000x reference)
import functools

import jax
import jax.numpy as jnp
from jax import lax
from jax.experimental import pallas as pl
from jax.experimental.pallas import tpu as pltpu
from jax.experimental.pallas import tpu_sc as plsc

N = 10000
E = 320000
D = 128

NC = 2
NS = 16
CHUNK = 128
NCHUNKS = E // CHUNK
CPT = 80
NCHUNKS_PAD = 2504

_HALVES = ((0, 48, 48),
           (1248, 52, 56))
_M = tuple(15 * CPT + last for (_, last, _l) in _HALVES)

ROWS_PER_TILE = 624
ROWS_REM = N - NS * ROWS_PER_TILE

_LOG2 = 0.6931471805599453


def _sc_mesh():
    return plsc.VectorSubcoreMesh(core_axis_name="c", subcore_axis_name="s",
                                  num_cores=NC, num_subcores=NS)


def _make_gather_body(base, last, last_load):
    def body(nodes_hbm, send2d, recv2d, sent_out, recv_out,
             idx_all, rows0, rows1, gsem0, gsem1, wsem0, wsem1):
        c = lax.axis_index("c")
        s = lax.axis_index("s")
        cb = s * CPT
        npairs = jnp.where(s == NS - 1, last // 2, CPT // 2)

        def run(idx2d_hbm, out_hbm):
            @pl.when(s < NS - 1)
            def _():
                pltpu.sync_copy(idx2d_hbm.at[pl.ds(base + cb, CPT)], idx_all)

            @pl.when(s == NS - 1)
            def _():
                pltpu.sync_copy(idx2d_hbm.at[pl.ds(base + cb, last_load)],
                                idx_all.at[pl.ds(0, last_load)])

            def gather_start(j, buf, sem):
                pltpu.async_copy(nodes_hbm.at[idx_all.at[j]], buf, sem)

            def gather_wait(j, buf, sem):
                pltpu.make_async_copy(nodes_hbm.at[idx_all.at[j]], buf,
                                      sem).wait()

            def write_start(j, buf, sem):
                pltpu.async_copy(
                    buf, out_hbm.at[pl.ds((cb + j) * CHUNK, CHUNK)], sem)

            def write_wait(sem):
                pltpu.make_async_copy(rows0, out_hbm.at[pl.ds(0, CHUNK)],
                                      sem).wait()

            def lbody(t, carry):
                j0 = 2 * t
                j1 = j0 + 1

                @pl.when(t > 0)
                def _():
                    write_wait(wsem0)

                gather_start(j0, rows0, gsem0)

                @pl.when(t > 0)
                def _():
                    write_wait(wsem1)

                gather_start(j1, rows1, gsem1)
                gather_wait(j0, rows0, gsem0)
                write_start(j0, rows0, wsem0)
                gather_wait(j1, rows1, gsem1)
                write_start(j1, rows1, wsem1)
                return carry

            lax.fori_loop(0, npairs, lbody, 0)
            write_wait(wsem0)
            write_wait(wsem1)

        @pl.when(c == 0)
        def _():
            run(send2d, sent_out)

        @pl.when(c == 1)
        def _():
            run(recv2d, recv_out)

    return body


@functools.cache
def _sc_gather_kernel(half):
    base, last, last_load = _HALVES[half]
    m_rows = _M[half] * CHUNK
    return pl.kernel(
        _make_gather_body(base, last, last_load),
        out_type=(
            jax.ShapeDtypeStruct((m_rows, D), jnp.float32),
            jax.ShapeDtypeStruct((m_rows, D), jnp.float32),
        ),
        mesh=_sc_mesh(),
        scratch_types=(
            pltpu.VMEM((CPT, CHUNK), jnp.int32),
            pltpu.VMEM((CHUNK, D), jnp.float32),
            pltpu.VMEM((CHUNK, D), jnp.float32),
            pltpu.SemaphoreType.DMA,
            pltpu.SemaphoreType.DMA,
            pltpu.SemaphoreType.DMA,
            pltpu.SemaphoreType.DMA,
        ),
    )


def _make_scatter_body(base, last):
    def body(wsend_hbm, wrecv_hbm, send_hbm, recv_hbm, zeros_hbm,
             sent_out, recv_out, idx0, idx1, rows0, rows1, acc,
             lsem0, lsem1, ssem0, ssem1):
        c = lax.axis_index("c")
        s = lax.axis_index("s")
        cb = s * CPT
        npairs = jnp.where(s == NS - 1, last // 2, CPT // 2)
        rbase = s * ROWS_PER_TILE

        pltpu.sync_copy(zeros_hbm.at[pl.ds(rbase, ROWS_PER_TILE)],
                        acc.at[pl.ds(rbase, ROWS_PER_TILE)])

        @pl.when(s == NS - 1)
        def _():
            pltpu.sync_copy(zeros_hbm.at[pl.ds(NS * ROWS_PER_TILE, ROWS_REM)],
                            acc.at[pl.ds(NS * ROWS_PER_TILE, ROWS_REM)])

        plsc.subcore_barrier()

        def run(idx_hbm, upd_hbm):
            def load_start(j, ibuf, rbuf, sem):
                pltpu.async_copy(
                    idx_hbm.at[pl.ds((base + cb + j) * CHUNK, CHUNK)], ibuf,
                    sem)
                pltpu.async_copy(
                    upd_hbm.at[pl.ds((cb + j) * CHUNK, CHUNK)], rbuf, sem)

            def load_wait(j, ibuf, rbuf, sem):
                pltpu.make_async_copy(
                    idx_hbm.at[pl.ds((base + cb + j) * CHUNK, CHUNK)], ibuf,
                    sem).wait()
                pltpu.make_async_copy(
                    upd_hbm.at[pl.ds((cb + j) * CHUNK, CHUNK)], rbuf,
                    sem).wait()

            def scat_start(ibuf, rbuf, sem):
                pltpu.async_copy(rbuf, acc.at[ibuf], sem, add=True)

            def scat_wait(sem):
                pltpu.make_async_copy(rows0, acc.at[idx0], sem).wait()

            def lbody(t, carry):
                j0 = 2 * t
                j1 = j0 + 1

                @pl.when(t > 0)
                def _():
                    scat_wait(ssem0)

                load_start(j0, idx0, rows0, lsem0)

                @pl.when(t > 0)
                def _():
                    scat_wait(ssem1)

                load_start(j1, idx1, rows1, lsem1)
                load_wait(j0, idx0, rows0, lsem0)
                scat_start(idx0, rows0, ssem0)
                load_wait(j1, idx1, rows1, lsem1)
                scat_start(idx1, rows1, ssem1)
                return carry

            lax.fori_loop(0, npairs, lbody, 0)
            scat_wait(ssem0)
            scat_wait(ssem1)

        @pl.when(c == 0)
        def _():
            run(send_hbm, wsend_hbm)

        @pl.when(c == 1)
        def _():
            run(recv_hbm, wrecv_hbm)

        plsc.subcore_barrier()

        def writeback(out_hbm):
            pltpu.sync_copy(acc.at[pl.ds(rbase, ROWS_PER_TILE)],
                            out_hbm.at[pl.ds(rbase, ROWS_PER_TILE)])

            @pl.when(s == NS - 1)
            def _():
                pltpu.sync_copy(
                    acc.at[pl.ds(NS * ROWS_PER_TILE, ROWS_REM)],
                    out_hbm.at[pl.ds(NS * ROWS_PER_TILE, ROWS_REM)])

        @pl.when(c == 0)
        def _():
            writeback(sent_out)

        @pl.when(c == 1)
        def _():
            writeback(recv_out)

    return body


@functools.cache
def _sc_scatter_kernel(half):
    base, last, _ = _HALVES[half]
    return pl.kernel(
        _make_scatter_body(base, last),
        out_type=(
            jax.ShapeDtypeStruct((N, D), jnp.float32),
            jax.ShapeDtypeStruct((N, D), jnp.float32),
        ),
        mesh=_sc_mesh(),
        scratch_types=(
            pltpu.VMEM((CHUNK,), jnp.int32),
            pltpu.VMEM((CHUNK,), jnp.int32),
            pltpu.VMEM((CHUNK, D), jnp.float32),
            pltpu.VMEM((CHUNK, D), jnp.float32),
            pltpu.VMEM_SHARED((N, D), jnp.float32),
            pltpu.SemaphoreType.DMA,
            pltpu.SemaphoreType.DMA,
            pltpu.SemaphoreType.DMA,
            pltpu.SemaphoreType.DMA,
        ),
    )


def _ssp(x):
    return jnp.maximum(x, 0.0) + jnp.log1p(jnp.exp(-jnp.abs(x))) - _LOG2


def _dot(a, b):
    return jax.lax.dot_general(a.astype(jnp.bfloat16), b.astype(jnp.bfloat16),
                               (((1,), (0,)), ((), ())),
                               preferred_element_type=jnp.float32)


def _mlp_ln(h, W2, b2, W3, b3, gamma, beta):
    h = _ssp(h)
    h = _ssp(_dot(h, W2) + b2)
    h = _dot(h, W3) + b3
    mu = jnp.mean(h, axis=1, keepdims=True)
    var = jnp.mean((h - mu) ** 2, axis=1, keepdims=True)
    h = (h - mu) * jax.lax.rsqrt(var + 1e-5)
    return h * gamma + beta


def _edge_block(e_ref, sa_ref, ra_ref,
                W1e_ref, W1s_ref, W1r_ref, b1_ref, W2_ref, b2_ref,
                W3_ref, b3_ref, g_ref, bt_ref,
                wr_ref, br_ref, ws_ref, bs_ref,
                new_e_ref, wsend_ref, wrecv_ref):
    x = e_ref[...]
    h = (_dot(x, W1e_ref[...]) + _dot(sa_ref[...], W1s_ref[...])
         + _dot(ra_ref[...], W1r_ref[...]) + b1_ref[...])
    u = _mlp_ln(h, W2_ref[...], b2_ref[...], W3_ref[...], b3_ref[...],
                g_ref[...], bt_ref[...])
    attn_r = jax.nn.sigmoid(
        jnp.sum(x * wr_ref[...], axis=1, keepdims=True) + br_ref[...])
    attn_s = jax.nn.sigmoid(
        jnp.sum(x * ws_ref[...], axis=1, keepdims=True) + bs_ref[...])
    new_e_ref[...] = x + u
    wsend_ref[...] = u * attn_s
    wrecv_ref[...] = u * attn_r


def _node_block(n_ref, ra0_ref, ra1_ref, sa0_ref, sa1_ref,
                W1n_ref, W1r_ref, W1s_ref, b1_ref, W2_ref, b2_ref,
                W3_ref, b3_ref, g_ref, bt_ref,
                out_ref):
    x = n_ref[...]
    ragg = ra0_ref[...] + ra1_ref[...]
    sagg = sa0_ref[...] + sa1_ref[...]
    h = (_dot(x, W1n_ref[...]) + _dot(ragg, W1r_ref[...])
         + _dot(sagg, W1s_ref[...]) + b1_ref[...])
    u = _mlp_ln(h, W2_ref[...], b2_ref[...], W3_ref[...], b3_ref[...],
                g_ref[...], bt_ref[...])
    out_ref[...] = x + u


def _full_spec(shape):
    return pl.BlockSpec(shape, lambda i: tuple(0 for _ in shape))


BE = 512
BN = 1000


def _edge_weights(p, wr, br, ws, bs):
    W1, W2, W3 = p["Ws"]
    b1, b2, b3 = p["bs"]
    row = lambda v: v.reshape(1, -1)
    return (W1[:D], W1[D:2 * D], W1[2 * D:], row(b1), W2, row(b2), W3,
            row(b3), row(p["gamma"]), row(p["beta"]),
            wr.reshape(1, D), br.reshape(1, 1), ws.reshape(1, D),
            bs.reshape(1, 1))


def _edge_call(half, edges, sent_attr, recv_attr, weights):
    base_blocks = (_HALVES[half][0] * CHUNK) // BE
    m_rows = _M[half] * CHUNK
    nblocks = m_rows // BE
    abs_spec = pl.BlockSpec((BE, D), lambda i: (i + base_blocks, 0))
    loc_spec = pl.BlockSpec((BE, D), lambda i: (i, 0))
    w_specs = [_full_spec(w.shape) for w in weights]
    out_shape = (
        jax.ShapeDtypeStruct((m_rows, D), jnp.float32),
        jax.ShapeDtypeStruct((m_rows, D), jnp.float32),
        jax.ShapeDtypeStruct((m_rows, D), jnp.float32),
    )
    in_specs = [abs_spec, loc_spec, loc_spec] + w_specs
    args = (edges, sent_attr, recv_attr) + weights
    return pl.pallas_call(
        _edge_block,
        grid=(nblocks,),
        in_specs=in_specs,
        out_specs=(loc_spec, loc_spec, loc_spec),
        out_shape=out_shape,
    )(*args)


def _node_call(nodes, ra0, ra1, sa0, sa1, p):
    W1, W2, W3 = p["Ws"]
    b1, b2, b3 = p["bs"]
    row = lambda v: v.reshape(1, -1)
    weights = (W1[:D], W1[D:2 * D], W1[2 * D:], row(b1), W2, row(b2), W3,
               row(b3), row(p["gamma"]), row(p["beta"]))
    w_specs = [_full_spec(w.shape) for w in weights]
    spec = pl.BlockSpec((BN, D), lambda i: (i, 0))
    return pl.pallas_call(
        _node_block,
        grid=(N // BN,),
        in_specs=[spec] * 5 + w_specs,
        out_specs=spec,
        out_shape=jax.ShapeDtypeStruct((N, D), jnp.float32),
    )(nodes, ra0, ra1, sa0, sa1, *weights)


def kernel(nodes, edges, senders, receivers, cutoff, params):
    del cutoff
    pad = ((0, NCHUNKS_PAD - NCHUNKS), (0, 0))
    senders2d = jnp.pad(senders.reshape(NCHUNKS, CHUNK), pad)
    receivers2d = jnp.pad(receivers.reshape(NCHUNKS, CHUNK), pad)
    zeros = jnp.zeros((N, D), jnp.float32)
    weights = _edge_weights(params["edge_mlp"], params["w_recv"],
                            params["b_recv"], params["w_send"],
                            params["b_send"])

    sent0, recv0 = _sc_gather_kernel(0)(nodes, senders2d, receivers2d)
    sent1, recv1 = _sc_gather_kernel(1)(nodes, senders2d, receivers2d)

    ne0, ws0, wr0 = _edge_call(0, edges, sent0, recv0, weights)
    sa0, ra0 = _sc_scatter_kernel(0)(ws0, wr0, senders, receivers, zeros)
    ne1, ws1, wr1 = _edge_call(1, edges, sent1, recv1, weights)
    sa1, ra1 = _sc_scatter_kernel(1)(ws1, wr1, senders, receivers, zeros)
    new_edges = jnp.concatenate([ne0, ne1], axis=0)

    new_nodes = _node_call(nodes, ra0, ra1, sa0, sa1, params["node_mlp"])
    return (new_nodes, new_edges)
